# baseline (speedup 1.0000x reference)
import jax
import jax.numpy as jnp
from jax import lax
from jax.experimental import pallas as pl
from jax.experimental.pallas import tpu as pltpu
from jax.experimental.pallas import tpu_sc as plsc

N = 50000
D = 128
H = 128
C = 48
ALPHA = 0.1
K = 10
A1, A2, A3 = 0.4, 0.3, 0.3
E = 1600000

NC = 2
NS = 16
LANES = 16

HNP = 25088
NPAD = 2 * HNP
TRASH = HNP

CHUNK = 128
SG = 8
NSG = 98
CPS = SG * NSG
RING = 4
NCHUNK = CPS * NS
EPAD = NCHUNK * CHUNK

FB = 56
FT = 28
RPT = FB * FT


def _sc_step_body(g_ref, e_ref, c1_ref, c0_ref, out_ref,
                  acc, ebuf, rows, fbuf, c1buf, c0buf,
                  isem, gs0, gs1, gs2, gs3, ss0, ss1, ss2, ss3):
    gsems = (gs0, gs1, gs2, gs3)
    ssems = (ss0, ss1, ss2, ss3)
    c = lax.axis_index("c")
    s = lax.axis_index("s")
    base = c * HNP
    cbase0 = s * CPS

    zero16 = jnp.zeros((LANES,), jnp.float32)
    for r in range(FB):
        for k in range(C // LANES):
            fbuf[r, pl.ds(k * LANES, LANES)] = zero16
    for t in range(FT):
        pltpu.sync_copy(fbuf, acc.at[pl.ds(s * RPT + t * FB, FB)])
    plsc.subcore_barrier()

    def idx_desc(par, sg):
        return pltpu.make_async_copy(
            e_ref.at[pl.ds(cbase0 + sg * SG, SG)],
            ebuf.at[pl.ds(par * SG, SG)], isem)

    def gdesc(row, b):
        return pltpu.make_async_copy(
            g_ref.at[ebuf.at[row, 0]], rows.at[b], gsems[b])

    def sdesc(row, b):
        return pltpu.make_async_copy(
            rows.at[b], acc.at[ebuf.at[row, 1]], ssems[b])

    d0 = idx_desc(0, 0)
    d0.start()
    d0.wait()

    def sg_body(sg, carry):
        par = lax.rem(sg, 2)

        @pl.when(sg > 0)
        def _():
            idx_desc(par, sg).wait()

        for j in range(SG):
            row = par * SG + j
            for v in range(CHUNK // LANES):
                dsl = pl.ds(v * LANES, LANES)
                dl = ebuf[row, 1, dsl] - base
                ok = (dl >= 0) & (dl < HNP)
                ebuf[row, 1, dsl] = jnp.where(ok, dl, TRASH)

        for cch in range(SG):
            b = cch % RING
            row = par * SG + cch
            if cch < RING:
                @pl.when(sg > 0)
                def _():
                    sdesc(row, b).wait()
            else:
                sdesc(row, b).wait()
            if cch == RING - 1:
                @pl.when(sg < NSG - 1)
                def _():
                    idx_desc(1 - par, sg + 1).start()
            gdesc(row, b).start()
            if cch >= 2:
                pb = (cch - 2) % RING
                gdesc(par * SG + cch - 2, pb).wait()
                sdesc(par * SG + cch - 2, pb).start(add=True)
        for cch in (SG - 2, SG - 1):
            b = cch % RING
            gdesc(par * SG + cch, b).wait()
            sdesc(par * SG + cch, b).start(add=True)
        return carry

    lax.fori_loop(0, NSG, sg_body, 0)
    for b in range(RING):
        sdesc(b, b).wait()
    plsc.subcore_barrier()

    def fin_body(t, carry):
        row0 = s * RPT + t * FB
        grow = base + row0
        pltpu.sync_copy(acc.at[pl.ds(row0, FB)], fbuf)
        pltpu.sync_copy(c1_ref.at[pl.ds(grow, FB)], c1buf)
        pltpu.sync_copy(c0_ref.at[pl.ds(grow, FB)], c0buf)
        for r in range(FB):
            for k in range(C // LANES):
                dsl = pl.ds(k * LANES, LANES)
                fbuf[r, dsl] = c1buf[r, dsl] * fbuf[r, dsl] + c0buf[r, dsl]
        pltpu.sync_copy(fbuf, out_ref.at[pl.ds(grow, FB)])
        return carry

    lax.fori_loop(0, FT, fin_body, 0)


_sc_step = pl.kernel(
    _sc_step_body,
    out_type=jax.ShapeDtypeStruct((NPAD, C), jnp.float32),
    mesh=plsc.VectorSubcoreMesh(core_axis_name="c", subcore_axis_name="s",
                                num_cores=NC, num_subcores=NS),
    scratch_types=[
        pltpu.VMEM_SHARED((HNP + 8, C), jnp.float32),
        pltpu.VMEM((2 * SG, 2, CHUNK), jnp.int32),
        pltpu.VMEM((RING, CHUNK, C), jnp.float32),
        pltpu.VMEM((FB, C), jnp.float32),
        pltpu.VMEM((FB, C), jnp.float32),
        pltpu.VMEM((FB, C), jnp.float32),
    ] + [pltpu.SemaphoreType.DMA] * 9,
    compiler_params=pltpu.CompilerParams(use_tc_tiling_on_sc=False),
)


BN = 128
TC_GRID = NPAD // BN


def _mlp_body(x1, x2, x3, w1, bb1, w2, bb2, w3, bb3, u1, u2, u3, bc, deg,
              g0_o, c1_o, c0_o, cf1_o, cf0_o):
    y1 = jnp.maximum(jnp.dot(x1[:], w1[:],
                             preferred_element_type=jnp.float32) + bb1[:], 0.0)
    y2 = jnp.maximum(jnp.dot(x2[:], w2[:],
                             preferred_element_type=jnp.float32) + bb2[:], 0.0)
    y3 = jnp.maximum(jnp.dot(x3[:], w3[:],
                             preferred_element_type=jnp.float32) + bb3[:], 0.0)
    hc = (jnp.dot(y1, u1[:], preferred_element_type=jnp.float32)
          + jnp.dot(y2, u2[:], preferred_element_type=jnp.float32)
          + jnp.dot(y3, u3[:], preferred_element_type=jnp.float32) + bc[:])
    norm = lax.rsqrt(jnp.maximum(deg[:], 1.0))
    g0 = norm * hc
    g0_o[:] = g0
    c1_o[:] = (1.0 - ALPHA) * norm * norm
    c0_o[:] = ALPHA * g0
    cf1_o[:] = (1.0 - ALPHA) * norm
    cf0_o[:] = ALPHA * hc


_row_spec = pl.BlockSpec((BN, D), lambda i: (i, 0))
_out_spec = pl.BlockSpec((BN, C), lambda i: (i, 0))
_full = lambda shape: pl.BlockSpec(shape, lambda i: (0,) * len(shape))

_mlp = pl.pallas_call(
    _mlp_body,
    grid=(TC_GRID,),
    in_specs=[
        _row_spec, _row_spec, _row_spec,
        _full((D, H)), _full((1, H)),
        _full((D, H)), _full((1, H)),
        _full((D, H)), _full((1, H)),
        _full((H, C)), _full((H, C)), _full((H, C)), _full((1, C)),
        _out_spec,
    ],
    out_specs=[_out_spec] * 5,
    out_shape=[jax.ShapeDtypeStruct((NPAD, C), jnp.float32)] * 5,
)


def kernel(features1, features2, features3, edge_index,
           W1a, b1a, W1b, b1b, W2a, b2a, W2b, b2b, W3a, b3a, W3b, b3b):
    src = edge_index[0]
    dst = edge_index[1]
    pad_e = EPAD - E
    src2d = jnp.concatenate(
        [src, jnp.zeros((pad_e,), jnp.int32)]).reshape(NCHUNK, CHUNK)
    dst2d = jnp.concatenate(
        [dst, jnp.full((pad_e,), NPAD, jnp.int32)]).reshape(NCHUNK, CHUNK)
    e3d = jnp.stack([src2d, dst2d], axis=1)

    ones2d = jnp.ones((NPAD, C), jnp.float32)
    zeros2d = jnp.zeros((NPAD, C), jnp.float32)
    deg48 = _sc_step(ones2d, e3d, ones2d, zeros2d)

    pad_n = NPAD - N
    f1 = jnp.concatenate([features1, jnp.zeros((pad_n, D), jnp.float32)])
    f2 = jnp.concatenate([features2, jnp.zeros((pad_n, D), jnp.float32)])
    f3 = jnp.concatenate([features3, jnp.zeros((pad_n, D), jnp.float32)])

    g0, c1, c0, cf1, cf0 = _mlp(
        f1, f2, f3,
        W1a, b1a.reshape(1, H), W2a, b2a.reshape(1, H), W3a, b3a.reshape(1, H),
        A1 * W1b, A2 * W2b, A3 * W3b,
        (A1 * b1b + A2 * b2b + A3 * b3b).reshape(1, C),
        deg48)

    g = g0
    for _ in range(K - 1):
        g = _sc_step(g, e3d, c1, c0)
    h = _sc_step(g, e3d, cf1, cf0)
    return h[:N]


# speedup vs baseline: 1.0167x; 1.0167x over previous
import jax
import jax.numpy as jnp
from jax import lax
from jax.experimental import pallas as pl
from jax.experimental.pallas import tpu as pltpu
from jax.experimental.pallas import tpu_sc as plsc

N = 50000
D = 128
H = 128
C = 48
ALPHA = 0.1
K = 10
A1, A2, A3 = 0.4, 0.3, 0.3
E = 1600000

NC = 2
NS = 16
LANES = 16

HNP = 25088
NPAD = 2 * HNP
TRASH = HNP

CHUNK = 128
SG = 8
NSG = 98
CPS = SG * NSG
RING = 4
NCHUNK = CPS * NS
EPAD = NCHUNK * CHUNK

FB = 56
FT = 28
RPT = FB * FT


def _sc_step_body(g_ref, e_ref, c1_ref, c0_ref, out_ref,
                  acc, ebuf, rows, fbuf, c1buf, c0buf,
                  isem, gs0, gs1, gs2, gs3, ss0, ss1, ss2, ss3):
    gsems = (gs0, gs1, gs2, gs3)
    ssems = (ss0, ss1, ss2, ss3)
    c = lax.axis_index("c")
    s = lax.axis_index("s")
    base = c * HNP
    cbase0 = s * CPS

    zero16 = jnp.zeros((LANES,), jnp.float32)
    for r in range(FB):
        for k in range(C // LANES):
            fbuf[r, pl.ds(k * LANES, LANES)] = zero16
    for t in range(FT):
        pltpu.sync_copy(fbuf, acc.at[pl.ds(s * RPT + t * FB, FB)])
    plsc.subcore_barrier()

    def idx_desc(par, sg):
        return pltpu.make_async_copy(
            e_ref.at[pl.ds(cbase0 + sg * SG, SG)],
            ebuf.at[pl.ds(par * SG, SG)], isem)

    def gdesc(row, b):
        return pltpu.make_async_copy(
            g_ref.at[ebuf.at[row, 0]], rows.at[b], gsems[b])

    def sdesc(row, b):
        return pltpu.make_async_copy(
            rows.at[b], acc.at[ebuf.at[row, 1]], ssems[b])

    d0 = idx_desc(0, 0)
    d0.start()
    d0.wait()

    def sg_body(sg, carry):
        par = lax.rem(sg, 2)

        @pl.when(sg > 0)
        def _():
            idx_desc(par, sg).wait()

        for j in range(SG):
            row = par * SG + j
            for v in range(CHUNK // LANES):
                dsl = pl.ds(v * LANES, LANES)
                dl = ebuf[row, 1, dsl] - base
                ok = (dl >= 0) & (dl < HNP)
                ebuf[row, 1, dsl] = jnp.where(ok, dl, TRASH)

        for cch in range(SG):
            b = cch % RING
            row = par * SG + cch
            if cch < RING:
                @pl.when(sg > 0)
                def _():
                    sdesc(row, b).wait()
            else:
                sdesc(row, b).wait()
            if cch == RING - 1:
                @pl.when(sg < NSG - 1)
                def _():
                    idx_desc(1 - par, sg + 1).start()
            gdesc(row, b).start()
            if cch >= 2:
                pb = (cch - 2) % RING
                gdesc(par * SG + cch - 2, pb).wait()
                sdesc(par * SG + cch - 2, pb).start(add=True)
        for cch in (SG - 2, SG - 1):
            b = cch % RING
            gdesc(par * SG + cch, b).wait()
            sdesc(par * SG + cch, b).start(add=True)
        return carry

    lax.fori_loop(0, NSG, sg_body, 0)
    for b in range(RING):
        sdesc(b, b).wait()
    plsc.subcore_barrier()

    def fin_body(t, carry):
        row0 = s * RPT + t * FB
        grow = base + row0
        pltpu.sync_copy(acc.at[pl.ds(row0, FB)], fbuf)
        pltpu.sync_copy(c1_ref.at[pl.ds(grow, FB)], c1buf)
        pltpu.sync_copy(c0_ref.at[pl.ds(grow, FB)], c0buf)
        for r in range(FB):
            for k in range(C // LANES):
                dsl = pl.ds(k * LANES, LANES)
                fbuf[r, dsl] = c1buf[r, dsl] * fbuf[r, dsl] + c0buf[r, dsl]
        pltpu.sync_copy(fbuf, out_ref.at[pl.ds(grow, FB)])
        return carry

    lax.fori_loop(0, FT, fin_body, 0)


_sc_step = pl.kernel(
    _sc_step_body,
    out_type=jax.ShapeDtypeStruct((NPAD, C), jnp.float32),
    mesh=plsc.VectorSubcoreMesh(core_axis_name="c", subcore_axis_name="s",
                                num_cores=NC, num_subcores=NS),
    scratch_types=[
        pltpu.VMEM_SHARED((HNP + 8, C), jnp.float32),
        pltpu.VMEM((2 * SG, 2, CHUNK), jnp.int32),
        pltpu.VMEM((RING, CHUNK, C), jnp.float32),
        pltpu.VMEM((FB, C), jnp.float32),
        pltpu.VMEM((FB, C), jnp.float32),
        pltpu.VMEM((FB, C), jnp.float32),
    ] + [pltpu.SemaphoreType.DMA] * 9,
    compiler_params=pltpu.CompilerParams(use_tc_tiling_on_sc=False),
)


DW = 16


def _sc_deg_body(e_ref, out_ref, acc, ebuf, ones_v, fbuf,
                 isem, ss0, ss1, ss2, ss3):
    ssems = (ss0, ss1, ss2, ss3)
    c = lax.axis_index("c")
    s = lax.axis_index("s")
    base = c * HNP
    cbase0 = s * CPS

    one16 = jnp.ones((LANES,), jnp.float32)
    zero16 = jnp.zeros((LANES,), jnp.float32)
    for r in range(CHUNK):
        ones_v[r, :] = one16
    for r in range(FB):
        fbuf[r, :] = zero16
    for t in range(FT):
        pltpu.sync_copy(fbuf, acc.at[pl.ds(s * RPT + t * FB, FB)])
    plsc.subcore_barrier()

    def idx_desc(par, sg):
        return pltpu.make_async_copy(
            e_ref.at[pl.ds(cbase0 + sg * SG, SG)],
            ebuf.at[pl.ds(par * SG, SG)], isem)

    def sdesc(row, b):
        return pltpu.make_async_copy(
            ones_v, acc.at[ebuf.at[row, 1]], ssems[b])

    d0 = idx_desc(0, 0)
    d0.start()
    d0.wait()

    def sg_body(sg, carry):
        par = lax.rem(sg, 2)

        @pl.when(sg > 0)
        def _():
            idx_desc(par, sg).wait()

        for j in range(SG):
            row = par * SG + j
            for v in range(CHUNK // LANES):
                dsl = pl.ds(v * LANES, LANES)
                dl = ebuf[row, 1, dsl] - base
                ok = (dl >= 0) & (dl < HNP)
                ebuf[row, 1, dsl] = jnp.where(ok, dl, TRASH)

        for cch in range(SG):
            b = cch % RING
            row = par * SG + cch
            if cch < RING:
                @pl.when(sg > 0)
                def _():
                    sdesc(row, b).wait()
            else:
                sdesc(row, b).wait()
            if cch == RING - 1:
                @pl.when(sg < NSG - 1)
                def _():
                    idx_desc(1 - par, sg + 1).start()
            sdesc(row, b).start(add=True)
        return carry

    lax.fori_loop(0, NSG, sg_body, 0)
    for b in range(RING):
        sdesc(b, b).wait()
    plsc.subcore_barrier()

    def fin_body(t, carry):
        row0 = s * RPT + t * FB
        pltpu.sync_copy(acc.at[pl.ds(row0, FB)],
                        out_ref.at[pl.ds(base + row0, FB)])
        return carry

    lax.fori_loop(0, FT, fin_body, 0)


_sc_deg = pl.kernel(
    _sc_deg_body,
    out_type=jax.ShapeDtypeStruct((NPAD, DW), jnp.float32),
    mesh=plsc.VectorSubcoreMesh(core_axis_name="c", subcore_axis_name="s",
                                num_cores=NC, num_subcores=NS),
    scratch_types=[
        pltpu.VMEM_SHARED((HNP + 8, DW), jnp.float32),
        pltpu.VMEM((2 * SG, 2, CHUNK), jnp.int32),
        pltpu.VMEM((CHUNK, DW), jnp.float32),
        pltpu.VMEM((FB, DW), jnp.float32),
    ] + [pltpu.SemaphoreType.DMA] * 5,
    compiler_params=pltpu.CompilerParams(use_tc_tiling_on_sc=False),
)


BN = 128
TC_GRID = NPAD // BN


def _mlp_body(x1, x2, x3, w1, bb1, w2, bb2, w3, bb3, u1, u2, u3, bc, deg,
              g0_o, c1_o, c0_o, cf1_o, cf0_o):
    y1 = jnp.maximum(jnp.dot(x1[:], w1[:],
                             preferred_element_type=jnp.float32) + bb1[:], 0.0)
    y2 = jnp.maximum(jnp.dot(x2[:], w2[:],
                             preferred_element_type=jnp.float32) + bb2[:], 0.0)
    y3 = jnp.maximum(jnp.dot(x3[:], w3[:],
                             preferred_element_type=jnp.float32) + bb3[:], 0.0)
    hc = (jnp.dot(y1, u1[:], preferred_element_type=jnp.float32)
          + jnp.dot(y2, u2[:], preferred_element_type=jnp.float32)
          + jnp.dot(y3, u3[:], preferred_element_type=jnp.float32) + bc[:])
    norm = lax.rsqrt(jnp.maximum(deg[:], 1.0))
    g0 = norm * hc
    g0_o[:] = g0
    c1_o[:] = (1.0 - ALPHA) * norm * norm
    c0_o[:] = ALPHA * g0
    cf1_o[:] = (1.0 - ALPHA) * norm
    cf0_o[:] = ALPHA * hc


_row_spec = pl.BlockSpec((BN, D), lambda i: (i, 0))
_out_spec = pl.BlockSpec((BN, C), lambda i: (i, 0))
_full = lambda shape: pl.BlockSpec(shape, lambda i: (0,) * len(shape))

_mlp = pl.pallas_call(
    _mlp_body,
    grid=(TC_GRID,),
    in_specs=[
        _row_spec, _row_spec, _row_spec,
        _full((D, H)), _full((1, H)),
        _full((D, H)), _full((1, H)),
        _full((D, H)), _full((1, H)),
        _full((H, C)), _full((H, C)), _full((H, C)), _full((1, C)),
        _out_spec,
    ],
    out_specs=[_out_spec] * 5,
    out_shape=[jax.ShapeDtypeStruct((NPAD, C), jnp.float32)] * 5,
)


def kernel(features1, features2, features3, edge_index,
           W1a, b1a, W1b, b1b, W2a, b2a, W2b, b2b, W3a, b3a, W3b, b3b):
    src = edge_index[0]
    dst = edge_index[1]
    pad_e = EPAD - E
    src2d = jnp.concatenate(
        [src, jnp.zeros((pad_e,), jnp.int32)]).reshape(NCHUNK, CHUNK)
    dst2d = jnp.concatenate(
        [dst, jnp.full((pad_e,), NPAD, jnp.int32)]).reshape(NCHUNK, CHUNK)
    e3d = jnp.stack([src2d, dst2d], axis=1)

    deg16 = _sc_deg(e3d)
    deg48 = jnp.concatenate([deg16, deg16, deg16], axis=1)

    pad_n = NPAD - N
    f1 = jnp.concatenate([features1, jnp.zeros((pad_n, D), jnp.float32)])
    f2 = jnp.concatenate([features2, jnp.zeros((pad_n, D), jnp.float32)])
    f3 = jnp.concatenate([features3, jnp.zeros((pad_n, D), jnp.float32)])

    g0, c1, c0, cf1, cf0 = _mlp(
        f1, f2, f3,
        W1a, b1a.reshape(1, H), W2a, b2a.reshape(1, H), W3a, b3a.reshape(1, H),
        A1 * W1b, A2 * W2b, A3 * W3b,
        (A1 * b1b + A2 * b2b + A3 * b3b).reshape(1, C),
        deg48)

    g = g0
    for _ in range(K - 1):
        g = _sc_step(g, e3d, c1, c0)
    h = _sc_step(g, e3d, cf1, cf0)
    return h[:N]
